# baseline (speedup 1.0000x reference)
import functools

import jax
import jax.numpy as jnp
from jax import lax
from jax.experimental import pallas as pl
from jax.experimental.pallas import tpu as pltpu
from jax.experimental.pallas import tpu_sc as plsc

D = 128
EB = 128
N_TILES = 16
N_CORES = 2
F = 6400
R = F + 128
ZB = 64
WP = 80


def _make_seg_kernel(e_pad, n_chunks):
    T = e_pad // N_TILES
    n_batches = T // EB
    n_cov = N_CORES * n_chunks * F
    zr = R // N_TILES
    rb = F // N_TILES
    mesh = plsc.VectorSubcoreMesh(core_axis_name="c", subcore_axis_name="s")

    @functools.partial(
        pl.kernel, mesh=mesh,
        out_type=(jax.ShapeDtypeStruct((n_cov, D), jnp.float32),
                  jax.ShapeDtypeStruct((n_cov,), jnp.float32)),
        scratch_types=[
            pltpu.VMEM((T,), jnp.int32),
            pltpu.VMEM((T,), jnp.int32),
            pltpu.VMEM((EB,), jnp.int32),
            pltpu.VMEM((EB,), jnp.int32),
            pltpu.VMEM((EB, D), jnp.float32),
            pltpu.VMEM((EB, D), jnp.float32),
            pltpu.VMEM((ZB, D), jnp.float32),
            pltpu.VMEM((WP, D), jnp.float32),
            pltpu.VMEM((rb,), jnp.float32),
            pltpu.VMEM_SHARED((R, D), jnp.float32),
        ],
    )
    def seg(xsrc, esrc, edst, agg_out, deg_out,
            esrc_v, edst_v, gi_v, si_v, rows_v, ones_v, zrow_v, wb_v, d1_v,
            acc_sh):
        c = lax.axis_index("c")
        s = lax.axis_index("s")
        tlo = s * T
        pltpu.sync_copy(esrc.at[pl.ds(tlo, T)], esrc_v)
        pltpu.sync_copy(edst.at[pl.ds(tlo, T)], edst_v)
        one16 = jnp.ones((16,), jnp.float32)
        zero16 = jnp.zeros((16,), jnp.float32)

        def fill(i, _):
            for g in range(D // 16):
                ones_v[i, pl.ds(g * 16, 16)] = one16
            return _
        lax.fori_loop(0, EB, fill, None)

        def zfill(i, _):
            for g in range(D // 16):
                zrow_v[i, pl.ds(g * 16, 16)] = zero16
            return _
        lax.fori_loop(0, ZB, zfill, None)

        def zero_acc():
            for p in range(zr // ZB):
                pltpu.sync_copy(zrow_v,
                                acc_sh.at[pl.ds(s * zr + p * ZB, ZB)])
            rem = zr % ZB
            if rem:
                pltpu.sync_copy(zrow_v.at[pl.ds(0, rem)],
                                acc_sh.at[pl.ds(s * zr + zr - rem, rem)])

        def feat_chunk(chunk, _):
            lo = (c * n_chunks + chunk) * F
            zero_acc()
            plsc.subcore_barrier()

            def batch(b, _):
                for g in range(EB // 16):
                    o = b * EB + g * 16
                    e_s = esrc_v[pl.ds(o, 16)]
                    e_d = edst_v[pl.ds(o, 16)]
                    m = (e_d >= lo) & (e_d < lo + F)
                    gi_v[pl.ds(g * 16, 16)] = e_s
                    si_v[pl.ds(g * 16, 16)] = jnp.where(m, e_d - lo, F)
                pltpu.sync_copy(xsrc.at[gi_v], rows_v)
                pltpu.sync_copy(rows_v, acc_sh.at[si_v], add=True)
                return _
            lax.fori_loop(0, n_batches, batch, None)
            plsc.subcore_barrier()
            pltpu.sync_copy(acc_sh.at[pl.ds(s * rb, rb)],
                            agg_out.at[pl.ds(lo + s * rb, rb)])
            plsc.subcore_barrier()
            return _

        lax.fori_loop(0, n_chunks, feat_chunk, None)

        def deg_chunk(chunk, _):
            lo = (c * n_chunks + chunk) * F
            zero_acc()
            plsc.subcore_barrier()

            def batch(b, _):
                for g in range(EB // 16):
                    o = b * EB + g * 16
                    e_d = edst_v[pl.ds(o, 16)]
                    m = (e_d >= lo) & (e_d < lo + F)
                    si_v[pl.ds(g * 16, 16)] = jnp.where(m, e_d - lo, F)
                pltpu.sync_copy(ones_v, acc_sh.at[si_v], add=True)
                return _
            lax.fori_loop(0, n_batches, batch, None)
            plsc.subcore_barrier()
            lanes = lax.iota(jnp.int32, 16)
            for p in range(rb // WP):
                pltpu.sync_copy(acc_sh.at[pl.ds(s * rb + p * WP, WP)], wb_v)

                def grp(j, _):
                    acc = jnp.zeros((16,), jnp.float32)
                    for i in range(16):
                        acc = jnp.where(lanes == i,
                                        wb_v[j * 16 + i, pl.ds(0, 16)], acc)
                    d1_v[pl.ds(p * WP + j * 16, 16)] = acc
                    return _
                lax.fori_loop(0, WP // 16, grp, None)
            pltpu.sync_copy(d1_v, deg_out.at[pl.ds(lo + s * rb, rb)])
            plsc.subcore_barrier()
            return _

        lax.fori_loop(0, n_chunks, deg_chunk, None)

    return seg


def _seg_sum(xsrc, esrc_p, edst_p, n_chunks):
    seg = _make_seg_kernel(esrc_p.shape[0], n_chunks)
    return seg(xsrc, esrc_p, edst_p)


def _pad_edges(esrc, edst):
    e = esrc.shape[0]
    unit = N_TILES * EB
    e_pad = ((e + unit - 1) // unit) * unit
    pad = e_pad - e
    if pad:
        esrc = jnp.concatenate([esrc, jnp.zeros((pad,), esrc.dtype)])
        edst = jnp.concatenate(
            [edst, jnp.full((pad,), 1 << 30, edst.dtype)])
    return esrc, edst


BLK = 1000


def _combine1(agg, deg, x, Wm, Ws, b):
    n = x.shape[0]

    def body(agg_ref, deg_ref, x_ref, wm_ref, ws_ref, b_ref, o_ref):
        r = 1.0 / jnp.maximum(deg_ref[...], 1.0)
        o_ref[...] = (
            jnp.dot(agg_ref[...] * r, wm_ref[...],
                    preferred_element_type=jnp.float32)
            + jnp.dot(x_ref[...], ws_ref[...],
                      preferred_element_type=jnp.float32)
            + b_ref[...])

    return pl.pallas_call(
        body,
        grid=(n // BLK,),
        in_specs=[
            pl.BlockSpec((BLK, D), lambda i: (i, 0)),
            pl.BlockSpec((BLK, 1), lambda i: (i, 0)),
            pl.BlockSpec((BLK, D), lambda i: (i, 0)),
            pl.BlockSpec((D, D), lambda i: (0, 0)),
            pl.BlockSpec((D, D), lambda i: (0, 0)),
            pl.BlockSpec((1, D), lambda i: (0, 0)),
        ],
        out_specs=pl.BlockSpec((BLK, D), lambda i: (i, 0)),
        out_shape=jax.ShapeDtypeStruct((n, D), jnp.float32),
    )(agg, deg.reshape(-1, 1), x, Wm, Ws, b.reshape(1, D))


def _combine2(agg1, deg1, Wm1, agg2, deg2, Wm2, x, Ws1, Ws2, b1, b2):
    n = x.shape[0]

    def body(a1_ref, d1_ref, wm1_ref, a2_ref, d2_ref, wm2_ref,
             x_ref, ws1_ref, ws2_ref, b_ref, o_ref):
        r1 = 1.0 / jnp.maximum(d1_ref[...], 1.0)
        r2 = 1.0 / jnp.maximum(d2_ref[...], 1.0)
        o_ref[...] = (
            jnp.dot(a1_ref[...] * r1, wm1_ref[...],
                    preferred_element_type=jnp.float32)
            + jnp.dot(a2_ref[...] * r2, wm2_ref[...],
                      preferred_element_type=jnp.float32)
            + jnp.dot(x_ref[...], ws1_ref[...] + ws2_ref[...],
                      preferred_element_type=jnp.float32)
            + b_ref[...])

    return pl.pallas_call(
        body,
        grid=(n // BLK,),
        in_specs=[
            pl.BlockSpec((BLK, D), lambda i: (i, 0)),
            pl.BlockSpec((BLK, 1), lambda i: (i, 0)),
            pl.BlockSpec((D, D), lambda i: (0, 0)),
            pl.BlockSpec((BLK, D), lambda i: (i, 0)),
            pl.BlockSpec((BLK, 1), lambda i: (i, 0)),
            pl.BlockSpec((D, D), lambda i: (0, 0)),
            pl.BlockSpec((BLK, D), lambda i: (i, 0)),
            pl.BlockSpec((D, D), lambda i: (0, 0)),
            pl.BlockSpec((D, D), lambda i: (0, 0)),
            pl.BlockSpec((1, D), lambda i: (0, 0)),
        ],
        out_specs=pl.BlockSpec((BLK, D), lambda i: (i, 0)),
        out_shape=jax.ShapeDtypeStruct((n, D), jnp.float32),
    )(agg1, deg1.reshape(-1, 1), Wm1, agg2, deg2.reshape(-1, 1), Wm2,
      x, Ws1, Ws2, (b1 + b2).reshape(1, D))


def kernel(x_review, x_product, x_customer,
           edge_review_to_product_src, edge_review_to_product_dst,
           edge_product_to_review_src, edge_product_to_review_dst,
           edge_review_to_customer_src, edge_review_to_customer_dst,
           edge_customer_to_review_src, edge_customer_to_review_dst,
           Wm_rp, Ws_rp, b_rp, Wm_pr, Ws_pr, b_pr,
           Wm_rc, Ws_rc, b_rc, Wm_cr, Ws_cr, b_cr,
           n_products, n_reviews, n_customers):
    rp_s, rp_d = _pad_edges(edge_review_to_product_src,
                            edge_review_to_product_dst)
    pr_s, pr_d = _pad_edges(edge_product_to_review_src,
                            edge_product_to_review_dst)
    rc_s, rc_d = _pad_edges(edge_review_to_customer_src,
                            edge_review_to_customer_dst)
    cr_s, cr_d = _pad_edges(edge_customer_to_review_src,
                            edge_customer_to_review_dst)

    agg_rp, deg_rp = _seg_sum(x_review, rp_s, rp_d, 1)
    agg_pr, deg_pr = _seg_sum(x_product, pr_s, pr_d, 8)
    agg_rc, deg_rc = _seg_sum(x_review, rc_s, rc_d, 4)
    agg_cr, deg_cr = _seg_sum(x_customer, cr_s, cr_d, 8)

    out_product = _combine1(agg_rp, deg_rp, x_product, Wm_rp, Ws_rp, b_rp)
    out_customer = _combine1(agg_rc, deg_rc, x_customer, Wm_rc, Ws_rc, b_rc)
    out_review = _combine2(agg_pr, deg_pr, Wm_pr, agg_cr, deg_cr, Wm_cr,
                           x_review, Ws_pr, Ws_cr, b_pr, b_cr)
    return (out_product, out_review, out_customer)


# speedup vs baseline: 1.0940x; 1.0940x over previous
import functools

import jax
import jax.numpy as jnp
from jax import lax
from jax.experimental import pallas as pl
from jax.experimental.pallas import tpu as pltpu
from jax.experimental.pallas import tpu_sc as plsc

D = 128
EB = 128
N_TILES = 16
N_CORES = 2
F = 8192
R = F + 128
ZB = 64
WP = 128


def _make_seg_kernel(e_pad, n_chunks):
    T = e_pad // N_TILES
    n_batches = T // EB
    n_cov = N_CORES * n_chunks * F
    zr = R // N_TILES
    rb = F // N_TILES
    mesh = plsc.VectorSubcoreMesh(core_axis_name="c", subcore_axis_name="s")

    @functools.partial(
        pl.kernel, mesh=mesh,
        out_type=(jax.ShapeDtypeStruct((n_cov, D), jnp.float32),
                  jax.ShapeDtypeStruct((n_cov,), jnp.float32)),
        scratch_types=[
            pltpu.VMEM((T,), jnp.int32),
            pltpu.VMEM((T,), jnp.int32),
            pltpu.VMEM((EB,), jnp.int32),
            pltpu.VMEM((EB,), jnp.int32),
            pltpu.VMEM((EB,), jnp.int32),
            pltpu.VMEM((EB,), jnp.int32),
            pltpu.VMEM((EB, D), jnp.float32),
            pltpu.VMEM((EB, D), jnp.float32),
            pltpu.VMEM((EB, D), jnp.float32),
            pltpu.VMEM((rb,), jnp.float32),
            pltpu.VMEM_SHARED((R, D), jnp.float32),
            pltpu.SemaphoreType.DMA,
            pltpu.SemaphoreType.DMA,
            pltpu.SemaphoreType.DMA,
            pltpu.SemaphoreType.DMA,
        ],
    )
    def seg(xsrc, esrc, edst, agg_out, deg_out,
            esrc_v, edst_v, gi0_v, si0_v, gi1_v, si1_v, rows0_v, rows1_v,
            ones_v, d1_v, acc_sh, gsem0, gsem1, ssem0, ssem1):
        c = lax.axis_index("c")
        s = lax.axis_index("s")
        tlo = s * T
        pltpu.sync_copy(esrc.at[pl.ds(tlo, T)], esrc_v)
        pltpu.sync_copy(edst.at[pl.ds(tlo, T)], edst_v)
        one16 = jnp.ones((16,), jnp.float32)
        zero16 = jnp.zeros((16,), jnp.float32)

        def fill(i, _):
            for g in range(D // 16):
                ones_v[i, pl.ds(g * 16, 16)] = one16
            return _
        lax.fori_loop(0, EB, fill, None)

        def zero_acc():
            def zfill(i, _):
                for g in range(D // 16):
                    rows1_v[i, pl.ds(g * 16, 16)] = zero16
                return _
            lax.fori_loop(0, ZB, zfill, None)
            zsrc = rows1_v.at[pl.ds(0, ZB)]
            for p in range(zr // ZB):
                pltpu.sync_copy(zsrc,
                                acc_sh.at[pl.ds(s * zr + p * ZB, ZB)])
            rem = zr % ZB
            if rem:
                pltpu.sync_copy(rows1_v.at[pl.ds(0, rem)],
                                acc_sh.at[pl.ds(s * zr + zr - rem, rem)])

        bufs = ((gi0_v, si0_v, rows0_v, gsem0, ssem0),
                (gi1_v, si1_v, rows1_v, gsem1, ssem1))

        def build_idx(b, lo, gi, si):
            for g in range(EB // 16):
                o = b * EB + g * 16
                e_s = esrc_v[pl.ds(o, 16)]
                e_d = edst_v[pl.ds(o, 16)]
                m = (e_d >= lo) & (e_d < lo + F)
                gi[pl.ds(g * 16, 16)] = e_s
                si[pl.ds(g * 16, 16)] = jnp.where(m, e_d - lo, F)

        def feat_chunk(chunk, _):
            lo = (c * n_chunks + chunk) * F
            zero_acc()
            plsc.subcore_barrier()

            gi, si, rows, gsem, _ss = bufs[0]
            build_idx(0, lo, gi, si)
            g_prev = pltpu.async_copy(xsrc.at[gi], rows, gsem)

            def pair(k, _):
                for j in range(2):
                    b = 2 * k + j
                    nxt = bufs[(j + 1) % 2]
                    cur = bufs[j % 2]

                    @pl.when(b + 1 < n_batches)
                    def _():
                        @pl.when(b + 1 >= 2)
                        def _():
                            pltpu.make_async_copy(
                                nxt[2], acc_sh.at[nxt[1]], nxt[4]).wait()
                        build_idx(b + 1, lo, nxt[0], nxt[1])
                        pltpu.async_copy(xsrc.at[nxt[0]], nxt[2], nxt[3])

                    @pl.when(b < n_batches)
                    def _():
                        pltpu.make_async_copy(
                            xsrc.at[cur[0]], cur[2], cur[3]).wait()
                        pltpu.async_copy(
                            cur[2], acc_sh.at[cur[1]], cur[4], add=True)
                return _

            lax.fori_loop(0, (n_batches + 1) // 2, pair, None)
            lb = (n_batches - 1) % 2
            pltpu.make_async_copy(
                bufs[lb][2], acc_sh.at[bufs[lb][1]], bufs[lb][4]).wait()
            @pl.when(n_batches >= 2)
            def _():
                ob = 1 - lb
                pltpu.make_async_copy(
                    bufs[ob][2], acc_sh.at[bufs[ob][1]], bufs[ob][4]).wait()
            plsc.subcore_barrier()
            pltpu.sync_copy(acc_sh.at[pl.ds(s * rb, rb)],
                            agg_out.at[pl.ds(lo + s * rb, rb)])
            plsc.subcore_barrier()
            return _

        lax.fori_loop(0, n_chunks, feat_chunk, None)

        def deg_chunk(chunk, _):
            lo = (c * n_chunks + chunk) * F
            zero_acc()
            plsc.subcore_barrier()

            def build_si(b, si):
                for g in range(EB // 16):
                    o = b * EB + g * 16
                    e_d = edst_v[pl.ds(o, 16)]
                    m = (e_d >= lo) & (e_d < lo + F)
                    si[pl.ds(g * 16, 16)] = jnp.where(m, e_d - lo, F)

            def pair_d(k, _):
                for j in range(2):
                    b = 2 * k + j
                    cur = bufs[j % 2]

                    @pl.when(b < n_batches)
                    def _():
                        @pl.when(b >= 2)
                        def _():
                            pltpu.make_async_copy(
                                ones_v, acc_sh.at[cur[1]], cur[4]).wait()
                        build_si(b, cur[1])
                        pltpu.async_copy(
                            ones_v, acc_sh.at[cur[1]], cur[4], add=True)
                return _

            lax.fori_loop(0, (n_batches + 1) // 2, pair_d, None)
            lb = (n_batches - 1) % 2
            pltpu.make_async_copy(
                ones_v, acc_sh.at[bufs[lb][1]], bufs[lb][4]).wait()
            @pl.when(n_batches >= 2)
            def _():
                ob = 1 - lb
                pltpu.make_async_copy(
                    ones_v, acc_sh.at[bufs[ob][1]], bufs[ob][4]).wait()
            plsc.subcore_barrier()
            lanes = lax.iota(jnp.int32, 16)
            for p in range(rb // WP):
                pltpu.sync_copy(acc_sh.at[pl.ds(s * rb + p * WP, WP)],
                                rows0_v)

                def grp(j, _):
                    acc = jnp.zeros((16,), jnp.float32)
                    for i in range(16):
                        acc = jnp.where(lanes == i,
                                        rows0_v[j * 16 + i, pl.ds(0, 16)],
                                        acc)
                    d1_v[pl.ds(p * WP + j * 16, 16)] = acc
                    return _
                lax.fori_loop(0, WP // 16, grp, None)
            pltpu.sync_copy(d1_v, deg_out.at[pl.ds(lo + s * rb, rb)])
            plsc.subcore_barrier()
            return _

        lax.fori_loop(0, n_chunks, deg_chunk, None)

    return seg


def _seg_sum(xsrc, esrc_p, edst_p, n_chunks):
    seg = _make_seg_kernel(esrc_p.shape[0], n_chunks)
    return seg(xsrc, esrc_p, edst_p)


def _pad_edges(esrc, edst):
    e = esrc.shape[0]
    unit = N_TILES * EB
    e_pad = ((e + unit - 1) // unit) * unit
    pad = e_pad - e
    if pad:
        esrc = jnp.concatenate([esrc, jnp.zeros((pad,), esrc.dtype)])
        edst = jnp.concatenate(
            [edst, jnp.full((pad,), 1 << 30, edst.dtype)])
    return esrc, edst


BLK = 1000


def _combine1(agg, deg, x, Wm, Ws, b):
    n = x.shape[0]

    def body(agg_ref, deg_ref, x_ref, wm_ref, ws_ref, b_ref, o_ref):
        r = 1.0 / jnp.maximum(deg_ref[...], 1.0)
        o_ref[...] = (
            jnp.dot(agg_ref[...] * r, wm_ref[...],
                    preferred_element_type=jnp.float32)
            + jnp.dot(x_ref[...], ws_ref[...],
                      preferred_element_type=jnp.float32)
            + b_ref[...])

    return pl.pallas_call(
        body,
        grid=(n // BLK,),
        in_specs=[
            pl.BlockSpec((BLK, D), lambda i: (i, 0)),
            pl.BlockSpec((BLK, 1), lambda i: (i, 0)),
            pl.BlockSpec((BLK, D), lambda i: (i, 0)),
            pl.BlockSpec((D, D), lambda i: (0, 0)),
            pl.BlockSpec((D, D), lambda i: (0, 0)),
            pl.BlockSpec((1, D), lambda i: (0, 0)),
        ],
        out_specs=pl.BlockSpec((BLK, D), lambda i: (i, 0)),
        out_shape=jax.ShapeDtypeStruct((n, D), jnp.float32),
    )(agg, deg.reshape(-1, 1), x, Wm, Ws, b.reshape(1, D))


def _combine2(agg1, deg1, Wm1, agg2, deg2, Wm2, x, Ws1, Ws2, b1, b2):
    n = x.shape[0]

    def body(a1_ref, d1_ref, wm1_ref, a2_ref, d2_ref, wm2_ref,
             x_ref, ws1_ref, ws2_ref, b_ref, o_ref):
        r1 = 1.0 / jnp.maximum(d1_ref[...], 1.0)
        r2 = 1.0 / jnp.maximum(d2_ref[...], 1.0)
        o_ref[...] = (
            jnp.dot(a1_ref[...] * r1, wm1_ref[...],
                    preferred_element_type=jnp.float32)
            + jnp.dot(a2_ref[...] * r2, wm2_ref[...],
                      preferred_element_type=jnp.float32)
            + jnp.dot(x_ref[...], ws1_ref[...] + ws2_ref[...],
                      preferred_element_type=jnp.float32)
            + b_ref[...])

    return pl.pallas_call(
        body,
        grid=(n // BLK,),
        in_specs=[
            pl.BlockSpec((BLK, D), lambda i: (i, 0)),
            pl.BlockSpec((BLK, 1), lambda i: (i, 0)),
            pl.BlockSpec((D, D), lambda i: (0, 0)),
            pl.BlockSpec((BLK, D), lambda i: (i, 0)),
            pl.BlockSpec((BLK, 1), lambda i: (i, 0)),
            pl.BlockSpec((D, D), lambda i: (0, 0)),
            pl.BlockSpec((BLK, D), lambda i: (i, 0)),
            pl.BlockSpec((D, D), lambda i: (0, 0)),
            pl.BlockSpec((D, D), lambda i: (0, 0)),
            pl.BlockSpec((1, D), lambda i: (0, 0)),
        ],
        out_specs=pl.BlockSpec((BLK, D), lambda i: (i, 0)),
        out_shape=jax.ShapeDtypeStruct((n, D), jnp.float32),
    )(agg1, deg1.reshape(-1, 1), Wm1, agg2, deg2.reshape(-1, 1), Wm2,
      x, Ws1, Ws2, (b1 + b2).reshape(1, D))


def kernel(x_review, x_product, x_customer,
           edge_review_to_product_src, edge_review_to_product_dst,
           edge_product_to_review_src, edge_product_to_review_dst,
           edge_review_to_customer_src, edge_review_to_customer_dst,
           edge_customer_to_review_src, edge_customer_to_review_dst,
           Wm_rp, Ws_rp, b_rp, Wm_pr, Ws_pr, b_pr,
           Wm_rc, Ws_rc, b_rc, Wm_cr, Ws_cr, b_cr,
           n_products, n_reviews, n_customers):
    rp_s, rp_d = _pad_edges(edge_review_to_product_src,
                            edge_review_to_product_dst)
    pr_s, pr_d = _pad_edges(edge_product_to_review_src,
                            edge_product_to_review_dst)
    rc_s, rc_d = _pad_edges(edge_review_to_customer_src,
                            edge_review_to_customer_dst)
    cr_s, cr_d = _pad_edges(edge_customer_to_review_src,
                            edge_customer_to_review_dst)

    agg_rp, deg_rp = _seg_sum(x_review, rp_s, rp_d, 1)
    agg_pr, deg_pr = _seg_sum(x_product, pr_s, pr_d, 7)
    agg_rc, deg_rc = _seg_sum(x_review, rc_s, rc_d, 4)
    agg_cr, deg_cr = _seg_sum(x_customer, cr_s, cr_d, 7)

    out_product = _combine1(agg_rp, deg_rp, x_product, Wm_rp, Ws_rp, b_rp)
    out_customer = _combine1(agg_rc, deg_rc, x_customer, Wm_rc, Ws_rc, b_rc)
    out_review = _combine2(agg_pr, deg_pr, Wm_pr, agg_cr, deg_cr, Wm_cr,
                           x_review, Ws_pr, Ws_cr, b_pr, b_cr)
    return (out_product, out_review, out_customer)
